# Initial kernel scaffold; baseline (speedup 1.0000x reference)
import jax
import jax.numpy as jnp
from jax.experimental import pallas as pl


def kernel(x, edge_index, W1, b1, Wmu, bmu, Wlv, blv):
    raise NotImplementedError("write your pallas kernel here")


# speedup vs baseline: 12.7362x; 12.7362x over previous
import functools

import jax
import jax.numpy as jnp
from jax import lax
from jax.experimental import pallas as pl
from jax.experimental.pallas import tpu as pltpu
from jax.experimental.pallas import tpu_sc as plsc

N = 10000
NP = 10240
E = 320000
D = 128
NC = 2
NS = 16
NW = NC * NS
EB = 128
KB = 80
EPAD = NW * KB * EB
ROWS_PER_SUB = NP // NS

_MESH = plsc.VectorSubcoreMesh(core_axis_name="c", subcore_axis_name="s")


def _sc_degree_body(dstm_hbm, deg_hbm, dst_v, ones_v, zero_v, sem, acc_sh):
    c = lax.axis_index("c")
    s = lax.axis_index("s")
    wid = c * NS + s

    @pl.loop(0, EB, step=16)
    def _(i):
        ones_v[pl.ds(i, 16)] = jnp.ones((16,), jnp.float32)

    @pl.loop(0, ROWS_PER_SUB, step=16)
    def _(i):
        zero_v[pl.ds(i, 16)] = jnp.zeros((16,), jnp.float32)

    base = s * ROWS_PER_SUB
    pltpu.sync_copy(zero_v, acc_sh.at[pl.ds(base, ROWS_PER_SUB)])
    plsc.subcore_barrier()

    pltpu.sync_copy(dstm_hbm.at[pl.ds(wid * KB, KB)], dst_v)

    @pl.loop(0, KB)
    def _(j):
        pltpu.sync_copy(ones_v, acc_sh.at[dst_v.at[j]], add=True)

    plsc.subcore_barrier()
    pltpu.sync_copy(acc_sh.at[pl.ds(base, ROWS_PER_SUB)],
                    deg_hbm.at[c, pl.ds(base, ROWS_PER_SUB)])


def _sc_degree(dstm):
    return pl.kernel(
        _sc_degree_body,
        out_type=jax.ShapeDtypeStruct((NC, NP), jnp.float32),
        mesh=_MESH,
        scratch_types=[
            pltpu.VMEM((KB, EB), jnp.int32),
            pltpu.VMEM((EB,), jnp.float32),
            pltpu.VMEM((ROWS_PER_SUB,), jnp.float32),
            pltpu.SemaphoreType.DMA,
            pltpu.VMEM_SHARED((NP,), jnp.float32),
        ],
    )(dstm)


def _sc_agg_body(y_hbm, srcm_hbm, dstm_hbm, z_hbm, src_v, dst_v, rows_v, sem,
                 acc_sh):
    c = lax.axis_index("c")
    s = lax.axis_index("s")
    wid = c * NS + s

    @pl.loop(0, EB)
    def _(i):
        @pl.loop(0, D, step=16)
        def _(j):
            rows_v[i, pl.ds(j, 16)] = jnp.zeros((16,), jnp.float32)

    base = s * ROWS_PER_SUB

    @pl.loop(0, ROWS_PER_SUB // EB)
    def _(k):
        pltpu.sync_copy(rows_v, acc_sh.at[pl.ds(base + k * EB, EB)])

    plsc.subcore_barrier()

    row0 = wid * KB
    pltpu.sync_copy(srcm_hbm.at[pl.ds(row0, KB)], src_v)
    pltpu.sync_copy(dstm_hbm.at[pl.ds(row0, KB)], dst_v)

    @pl.loop(0, KB)
    def _(j):
        pltpu.async_copy(y_hbm.at[src_v.at[j]], rows_v, sem).wait()
        pltpu.sync_copy(rows_v, acc_sh.at[dst_v.at[j]], add=True)

    plsc.subcore_barrier()
    pltpu.sync_copy(acc_sh.at[pl.ds(base, ROWS_PER_SUB)],
                    z_hbm.at[c, pl.ds(base, ROWS_PER_SUB)])


def _sc_agg(y, srcm, dstm):
    return pl.kernel(
        _sc_agg_body,
        out_type=jax.ShapeDtypeStruct((NC, NP, D), jnp.float32),
        mesh=_MESH,
        scratch_types=[
            pltpu.VMEM((KB, EB), jnp.int32),
            pltpu.VMEM((KB, EB), jnp.int32),
            pltpu.VMEM((EB, D), jnp.float32),
            pltpu.SemaphoreType.DMA,
            pltpu.VMEM_SHARED((NP, D), jnp.float32),
        ],
    )(y, srcm, dstm)


def _dinv(deg_ref):
    deg = deg_ref[0] + deg_ref[1] + 1.0
    return lax.rsqrt(jnp.maximum(deg, 1.0))


def _tc_prescale_body(deg_ref, x_ref, y_ref):
    y_ref[...] = x_ref[...] * _dinv(deg_ref)[:, None]


def _tc_prescale(deg_p, x_p):
    blk = 1024
    return pl.pallas_call(
        _tc_prescale_body,
        grid=(NP // blk,),
        in_specs=[
            pl.BlockSpec((NC, blk), lambda i: (0, i)),
            pl.BlockSpec((blk, D), lambda i: (i, 0)),
        ],
        out_specs=pl.BlockSpec((blk, D), lambda i: (i, 0)),
        out_shape=jax.ShapeDtypeStruct((NP, D), jnp.float32),
    )(deg_p, x_p)


def _tc_mid_body(z_ref, y_ref, deg_ref, w1_ref, b1_ref, w2_ref, out_ref):
    dinv = _dinv(deg_ref)[:, None]
    agg = (z_ref[0] + z_ref[1] + y_ref[...]) * dinv
    h = jnp.dot(agg, w1_ref[...], preferred_element_type=jnp.float32,
                precision=lax.Precision.HIGHEST)
    h = jnp.maximum(h + b1_ref[...], 0.0)
    h2 = jnp.dot(h, w2_ref[...], preferred_element_type=jnp.float32,
                 precision=lax.Precision.HIGHEST)
    out_ref[...] = h2 * dinv


def _tc_mid(z_p, y, deg_p, W1, b1r, W2):
    blk = 1024
    return pl.pallas_call(
        _tc_mid_body,
        grid=(NP // blk,),
        in_specs=[
            pl.BlockSpec((NC, blk, D), lambda i: (0, i, 0)),
            pl.BlockSpec((blk, D), lambda i: (i, 0)),
            pl.BlockSpec((NC, blk), lambda i: (0, i)),
            pl.BlockSpec((128, 256), lambda i: (0, 0)),
            pl.BlockSpec((1, 256), lambda i: (0, 0)),
            pl.BlockSpec((256, D), lambda i: (0, 0)),
        ],
        out_specs=pl.BlockSpec((blk, D), lambda i: (i, 0)),
        out_shape=jax.ShapeDtypeStruct((NP, D), jnp.float32),
    )(z_p, y, deg_p, W1, b1r, W2)


def _tc_final_body(z_ref, y_ref, deg_ref, bmu_ref, blv_ref, mu_ref, lv_ref):
    dinv = _dinv(deg_ref)[:, None]
    o = (z_ref[0] + z_ref[1] + y_ref[...]) * dinv
    mu_ref[...] = o[:, :64] + bmu_ref[...]
    lv_ref[...] = o[:, 64:] + blv_ref[...]


def _tc_final(z2_p, y2, deg_p, bmur, blvr):
    blk = 1024
    return pl.pallas_call(
        _tc_final_body,
        grid=(NP // blk,),
        in_specs=[
            pl.BlockSpec((NC, blk, D), lambda i: (0, i, 0)),
            pl.BlockSpec((blk, D), lambda i: (i, 0)),
            pl.BlockSpec((NC, blk), lambda i: (0, i)),
            pl.BlockSpec((1, 64), lambda i: (0, 0)),
            pl.BlockSpec((1, 64), lambda i: (0, 0)),
        ],
        out_specs=[
            pl.BlockSpec((blk, 64), lambda i: (i, 0)),
            pl.BlockSpec((blk, 64), lambda i: (i, 0)),
        ],
        out_shape=[
            jax.ShapeDtypeStruct((NP, 64), jnp.float32),
            jax.ShapeDtypeStruct((NP, 64), jnp.float32),
        ],
    )(z2_p, y2, deg_p, bmur, blvr)


@jax.jit
def kernel(x, edge_index, W1, b1, Wmu, bmu, Wlv, blv):
    src = edge_index[0].astype(jnp.int32)
    dst = edge_index[1].astype(jnp.int32)
    pad = jnp.full((EPAD - E,), N, jnp.int32)
    srcm = jnp.concatenate([src, pad]).reshape(EPAD // EB, EB)
    dstm = jnp.concatenate([dst, pad]).reshape(EPAD // EB, EB)

    deg_p = _sc_degree(dstm)
    x_p = jnp.pad(x, ((0, NP - N), (0, 0)))
    y = _tc_prescale(deg_p, x_p)
    z_p = _sc_agg(y, srcm, dstm)
    W2 = jnp.concatenate([Wmu, Wlv], axis=1)
    y2 = _tc_mid(z_p, y, deg_p, W1, b1.reshape(1, -1), W2)
    z2_p = _sc_agg(y2, srcm, dstm)
    mu, lv = _tc_final(z2_p, y2, deg_p, bmu.reshape(1, -1), blv.reshape(1, -1))
    return (mu[:N], lv[:N])


# speedup vs baseline: 14.0281x; 1.1014x over previous
import functools

import jax
import jax.numpy as jnp
from jax import lax
from jax.experimental import pallas as pl
from jax.experimental.pallas import tpu as pltpu
from jax.experimental.pallas import tpu_sc as plsc

N = 10000
NP = 10240
E = 320000
D = 128
NC = 2
NS = 16
NW = NC * NS
EB = 128
KB = 80
CB = 8
EPAD = NW * KB * EB
ROWS_PER_SUB = NP // NS

_MESH = plsc.VectorSubcoreMesh(core_axis_name="c", subcore_axis_name="s")


def _sc_degree_body(dstm_hbm, deg_hbm, dst_v, ones_v, zero_v, sem, acc_sh):
    c = lax.axis_index("c")
    s = lax.axis_index("s")
    wid = c * NS + s

    @pl.loop(0, EB, step=16)
    def _(i):
        ones_v[pl.ds(i, 16)] = jnp.ones((16,), jnp.float32)

    @pl.loop(0, ROWS_PER_SUB, step=16)
    def _(i):
        zero_v[pl.ds(i, 16)] = jnp.zeros((16,), jnp.float32)

    base = s * ROWS_PER_SUB
    pltpu.sync_copy(zero_v, acc_sh.at[pl.ds(base, ROWS_PER_SUB)])
    plsc.subcore_barrier()

    pltpu.sync_copy(dstm_hbm.at[pl.ds(wid * KB, KB)], dst_v)

    @pl.loop(0, KB)
    def _(j):
        pltpu.sync_copy(ones_v, acc_sh.at[dst_v.at[j]], add=True)

    plsc.subcore_barrier()
    pltpu.sync_copy(acc_sh.at[pl.ds(base, ROWS_PER_SUB)],
                    deg_hbm.at[c, pl.ds(base, ROWS_PER_SUB)])


def _sc_degree(dstm):
    return pl.kernel(
        _sc_degree_body,
        out_type=jax.ShapeDtypeStruct((NC, NP), jnp.float32),
        mesh=_MESH,
        scratch_types=[
            pltpu.VMEM((KB, EB), jnp.int32),
            pltpu.VMEM((EB,), jnp.float32),
            pltpu.VMEM((ROWS_PER_SUB,), jnp.float32),
            pltpu.SemaphoreType.DMA,
            pltpu.VMEM_SHARED((NP,), jnp.float32),
        ],
    )(dstm)


def _sc_agg_body(y_hbm, srcm_hbm, dstm_hbm, z_hbm, src_v, dst_v, rows_a,
                 rows_b, sem_a, sem_b, acc_sh):
    c = lax.axis_index("c")
    s = lax.axis_index("s")
    wid = c * NS + s

    @pl.loop(0, EB)
    def _(i):
        @pl.loop(0, D, step=16)
        def _(j):
            rows_a[i, pl.ds(j, 16)] = jnp.zeros((16,), jnp.float32)

    base = s * ROWS_PER_SUB

    @pl.loop(0, ROWS_PER_SUB // EB)
    def _(k):
        pltpu.sync_copy(rows_a, acc_sh.at[pl.ds(base + k * EB, EB)])

    plsc.subcore_barrier()

    def _fire(j, buf, sem):
        pltpu.async_copy(y_hbm.at[src_v.at[j]], buf, sem)

    def _drain(j, buf, sem):
        pltpu.make_async_copy(y_hbm.at[src_v.at[j]], buf, sem).wait()

    @pl.loop(0, KB // CB)
    def _(ci):
        row0 = wid * KB + ci * CB
        pltpu.sync_copy(srcm_hbm.at[pl.ds(row0, CB)], src_v)
        pltpu.sync_copy(dstm_hbm.at[pl.ds(row0, CB)], dst_v)

        _fire(0, rows_a, sem_a)
        _fire(1, rows_b, sem_b)

        @pl.loop(0, CB, step=2)
        def _(j):
            _drain(j, rows_a, sem_a)
            pltpu.sync_copy(rows_a, acc_sh.at[dst_v.at[j]], add=True)

            @pl.when(j + 2 < CB)
            def _():
                _fire(j + 2, rows_a, sem_a)

            _drain(j + 1, rows_b, sem_b)
            pltpu.sync_copy(rows_b, acc_sh.at[dst_v.at[j + 1]], add=True)

            @pl.when(j + 3 < CB)
            def _():
                _fire(j + 3, rows_b, sem_b)

    plsc.subcore_barrier()
    pltpu.sync_copy(acc_sh.at[pl.ds(base, ROWS_PER_SUB)],
                    z_hbm.at[c, pl.ds(base, ROWS_PER_SUB)])


def _sc_agg(y, srcm, dstm):
    return pl.kernel(
        _sc_agg_body,
        out_type=jax.ShapeDtypeStruct((NC, NP, D), jnp.float32),
        mesh=_MESH,
        scratch_types=[
            pltpu.VMEM((CB, EB), jnp.int32),
            pltpu.VMEM((CB, EB), jnp.int32),
            pltpu.VMEM((EB, D), jnp.float32),
            pltpu.VMEM((EB, D), jnp.float32),
            pltpu.SemaphoreType.DMA,
            pltpu.SemaphoreType.DMA,
            pltpu.VMEM_SHARED((NP, D), jnp.float32),
        ],
    )(y, srcm, dstm)


def _dinv(deg_ref):
    deg = deg_ref[0] + deg_ref[1] + 1.0
    return lax.rsqrt(jnp.maximum(deg, 1.0))


def _tc_prescale_body(deg_ref, x_ref, y_ref):
    y_ref[...] = x_ref[...] * _dinv(deg_ref)[:, None]


def _tc_prescale(deg_p, x_p):
    blk = 1024
    return pl.pallas_call(
        _tc_prescale_body,
        grid=(NP // blk,),
        in_specs=[
            pl.BlockSpec((NC, blk), lambda i: (0, i)),
            pl.BlockSpec((blk, D), lambda i: (i, 0)),
        ],
        out_specs=pl.BlockSpec((blk, D), lambda i: (i, 0)),
        out_shape=jax.ShapeDtypeStruct((NP, D), jnp.float32),
    )(deg_p, x_p)


def _tc_mid_body(z_ref, y_ref, deg_ref, w1_ref, b1_ref, w2_ref, out_ref):
    dinv = _dinv(deg_ref)[:, None]
    agg = (z_ref[0] + z_ref[1] + y_ref[...]) * dinv
    h = jnp.dot(agg, w1_ref[...], preferred_element_type=jnp.float32,
                precision=lax.Precision.HIGHEST)
    h = jnp.maximum(h + b1_ref[...], 0.0)
    h2 = jnp.dot(h, w2_ref[...], preferred_element_type=jnp.float32,
                 precision=lax.Precision.HIGHEST)
    out_ref[...] = h2 * dinv


def _tc_mid(z_p, y, deg_p, W1, b1r, W2):
    blk = 1024
    return pl.pallas_call(
        _tc_mid_body,
        grid=(NP // blk,),
        in_specs=[
            pl.BlockSpec((NC, blk, D), lambda i: (0, i, 0)),
            pl.BlockSpec((blk, D), lambda i: (i, 0)),
            pl.BlockSpec((NC, blk), lambda i: (0, i)),
            pl.BlockSpec((128, 256), lambda i: (0, 0)),
            pl.BlockSpec((1, 256), lambda i: (0, 0)),
            pl.BlockSpec((256, D), lambda i: (0, 0)),
        ],
        out_specs=pl.BlockSpec((blk, D), lambda i: (i, 0)),
        out_shape=jax.ShapeDtypeStruct((NP, D), jnp.float32),
    )(z_p, y, deg_p, W1, b1r, W2)


def _tc_final_body(z_ref, y_ref, deg_ref, bmu_ref, blv_ref, mu_ref, lv_ref):
    dinv = _dinv(deg_ref)[:, None]
    o = (z_ref[0] + z_ref[1] + y_ref[...]) * dinv
    mu_ref[...] = o[:, :64] + bmu_ref[...]
    lv_ref[...] = o[:, 64:] + blv_ref[...]


def _tc_final(z2_p, y2, deg_p, bmur, blvr):
    blk = 1024
    return pl.pallas_call(
        _tc_final_body,
        grid=(NP // blk,),
        in_specs=[
            pl.BlockSpec((NC, blk, D), lambda i: (0, i, 0)),
            pl.BlockSpec((blk, D), lambda i: (i, 0)),
            pl.BlockSpec((NC, blk), lambda i: (0, i)),
            pl.BlockSpec((1, 64), lambda i: (0, 0)),
            pl.BlockSpec((1, 64), lambda i: (0, 0)),
        ],
        out_specs=[
            pl.BlockSpec((blk, 64), lambda i: (i, 0)),
            pl.BlockSpec((blk, 64), lambda i: (i, 0)),
        ],
        out_shape=[
            jax.ShapeDtypeStruct((NP, 64), jnp.float32),
            jax.ShapeDtypeStruct((NP, 64), jnp.float32),
        ],
    )(z2_p, y2, deg_p, bmur, blvr)


@jax.jit
def kernel(x, edge_index, W1, b1, Wmu, bmu, Wlv, blv):
    src = edge_index[0].astype(jnp.int32)
    dst = edge_index[1].astype(jnp.int32)
    pad = jnp.full((EPAD - E,), N, jnp.int32)
    srcm = jnp.concatenate([src, pad]).reshape(EPAD // EB, EB)
    dstm = jnp.concatenate([dst, pad]).reshape(EPAD // EB, EB)

    deg_p = _sc_degree(dstm)
    x_p = jnp.pad(x, ((0, NP - N), (0, 0)))
    y = _tc_prescale(deg_p, x_p)
    z_p = _sc_agg(y, srcm, dstm)
    W2 = jnp.concatenate([Wmu, Wlv], axis=1)
    y2 = _tc_mid(z_p, y, deg_p, W1, b1.reshape(1, -1), W2)
    z2_p = _sc_agg(y2, srcm, dstm)
    mu, lv = _tc_final(z2_p, y2, deg_p, bmu.reshape(1, -1), blv.reshape(1, -1))
    return (mu[:N], lv[:N])
